# baseline (speedup 1.0000x reference)
import jax
import jax.numpy as jnp
from jax import lax
from jax.experimental import pallas as pl
from jax.experimental.pallas import tpu as pltpu
from jax.experimental.pallas import tpu_sc as plsc

V_DIM = 1000000
EMB = 32
BATCH = 16384
CTX = 20
RPL = 4
NC, NS = 2, 16
NW = NC * NS
S_PER_W = BATCH // NW
CHUNK = 32
N_CHUNKS = S_PER_W // CHUNK
ROWS = CHUNK * CTX
OUT_LINES = CHUNK // RPL
GROUPS = CHUNK // 16
INV_CTX = float(1.0 / CTX)


def _sc_body(idx_hbm, off_hbm, tab_hbm, out_hbm, idx_v, off_v, rows_v,
             out_v, sem):
    wid = lax.axis_index("s") * NC + lax.axis_index("c")
    base_out = wid * S_PER_W
    lanes = lax.iota(jnp.int32, 16)
    orow = lanes // RPL
    ocol0 = (lanes % RPL) * EMB

    def chunk_body(ci, carry):
        off_out = base_out + ci * CHUNK
        off_idx = pl.multiple_of(off_out * CTX, 128)
        pltpu.sync_copy(idx_hbm.at[pl.ds(off_idx, ROWS)], idx_v)
        pltpu.sync_copy(off_hbm.at[pl.ds(off_idx, ROWS)], off_v)
        pltpu.async_copy(tab_hbm.at[idx_v], rows_v, sem).wait()

        for g in range(GROUPS):
            row0 = g * 16 * CTX + lanes * CTX
            accs = [jnp.zeros((16,), jnp.float32) for _ in range(EMB)]
            for c in range(CTX):
                rowv = row0 + c
                starts = plsc.load_gather(off_v, [rowv])
                for d in range(EMB):
                    v = plsc.load_gather(rows_v, [rowv, starts + d])
                    accs[d] = accs[d] + plsc.bitcast(v, jnp.float32)
            for d in range(EMB):
                plsc.store_scatter(
                    out_v, [g * (16 // RPL) + orow, ocol0 + d],
                    plsc.bitcast(accs[d] * INV_CTX, jnp.int32))

        out_line = pl.multiple_of(off_out // RPL, 8)
        pltpu.sync_copy(out_v, out_hbm.at[pl.ds(out_line, OUT_LINES)])
        return carry

    lax.fori_loop(0, N_CHUNKS, chunk_body, 0)


@jax.jit
def _cbow(x_flat, tab):
    tab_lines = lax.bitcast_convert_type(tab, jnp.int32).reshape(
        V_DIM // RPL, 128)
    idx = lax.shift_right_logical(x_flat, 2)
    off = lax.mul(lax.bitwise_and(x_flat, 3), EMB)
    mesh = plsc.VectorSubcoreMesh(core_axis_name="c", subcore_axis_name="s")
    f = pl.kernel(
        _sc_body,
        out_type=jax.ShapeDtypeStruct((BATCH // RPL, 128), jnp.int32),
        mesh=mesh,
        scratch_types=[
            pltpu.VMEM((ROWS,), jnp.int32),
            pltpu.VMEM((ROWS,), jnp.int32),
            pltpu.VMEM((ROWS, 128), jnp.int32),
            pltpu.VMEM((OUT_LINES, 128), jnp.int32),
            pltpu.SemaphoreType.DMA,
        ],
        compiler_params=pltpu.CompilerParams(needs_layout_passes=False),
    )
    out_lines = f(idx, off, tab_lines)
    return lax.bitcast_convert_type(
        out_lines.reshape(BATCH, EMB), jnp.float32)


def kernel(x, emb_weight):
    return _cbow(x.reshape(-1), emb_weight)


# speedup vs baseline: 1.5305x; 1.5305x over previous
import jax
import jax.numpy as jnp
from jax import lax
from jax.experimental import pallas as pl
from jax.experimental.pallas import tpu as pltpu
from jax.experimental.pallas import tpu_sc as plsc

V_DIM = 1000000
EMB = 32
BATCH = 16384
CTX = 20
NC, NS = 2, 16
NW = NC * NS
S_PER_W = BATCH // NW
CHUNK = 128
N_CHUNKS = S_PER_W // CHUNK
ROWS = CHUNK * CTX
INV_CTX = float(1.0 / CTX)


def _sc_body(x_hbm, tab_hbm, out_hbm, idx2_v, idx_v, rows_v, out_v, sem):
    wid = lax.axis_index("s") * NC + lax.axis_index("c")
    base_out = wid * S_PER_W

    def chunk_body(ci, carry):
        off_out = base_out + ci * CHUNK
        pltpu.sync_copy(x_hbm.at[pl.ds(off_out, CHUNK)], idx2_v)

        def repack_body(o, c2):
            idx_v[pl.ds(o * CTX, 16)] = idx2_v[o, pl.ds(0, 16)]
            idx_v[pl.ds(o * CTX + CTX - 16, 16)] = idx2_v[
                o, pl.ds(CTX - 16, 16)]
            return c2

        lax.fori_loop(0, CHUNK, repack_body, 0)
        pltpu.async_copy(tab_hbm.at[idx_v], rows_v, sem).wait()

        def out_body(o, c2):
            base = o * CTX
            for h in range(EMB // 16):
                sl = pl.ds(h * 16, 16)
                vals = [rows_v[base + c, sl] for c in range(CTX)]
                while len(vals) > 1:
                    vals = [a + b for a, b in zip(vals[::2], vals[1::2])] + (
                        [vals[-1]] if len(vals) % 2 else [])
                out_v[o, sl] = vals[0] * INV_CTX
            return c2

        lax.fori_loop(0, CHUNK, out_body, 0)
        pltpu.sync_copy(out_v, out_hbm.at[pl.ds(off_out, CHUNK)])
        return carry

    lax.fori_loop(0, N_CHUNKS, chunk_body, 0)


@jax.jit
def _cbow(x, tab):
    mesh = plsc.VectorSubcoreMesh(core_axis_name="c", subcore_axis_name="s")
    f = pl.kernel(
        _sc_body,
        out_type=jax.ShapeDtypeStruct((BATCH, EMB), jnp.float32),
        mesh=mesh,
        scratch_types=[
            pltpu.VMEM((CHUNK, CTX), jnp.int32),
            pltpu.VMEM((ROWS,), jnp.int32),
            pltpu.VMEM((ROWS, EMB), jnp.float32),
            pltpu.VMEM((CHUNK, EMB), jnp.float32),
            pltpu.SemaphoreType.DMA,
        ],
        compiler_params=pltpu.CompilerParams(use_tc_tiling_on_sc=False),
    )
    return f(x, tab)


def kernel(x, emb_weight):
    return _cbow(x, emb_weight)
